# baseline (speedup 1.0000x reference)
import functools

import jax
import jax.numpy as jnp
from jax import lax
from jax.experimental import pallas as pl
from jax.experimental.pallas import tpu as pltpu
from jax.experimental.pallas import tpu_sc as plsc

B, C, T = 16, 8, 2048
QUANT_LEVELS, QUANT_EMB = 1024, 64
NUM_CLASSES, CLASS_EMB = 1000, 128

NW = 32
ROWS_OUT = B * T * C
ROWS_COND = B * T
PAIRS_PER_W = B * C // NW
COND_PER_W = ROWS_COND // NW
OUT_CHUNK = 1024
L = 16

_mesh = plsc.VectorSubcoreMesh(core_axis_name="c", subcore_axis_name="s")


@functools.partial(
    pl.kernel,
    mesh=_mesh,
    out_type=jax.ShapeDtypeStruct((ROWS_OUT, QUANT_EMB), jnp.float32),
    scratch_types=[
        pltpu.VMEM((OUT_CHUNK // 128, 128), jnp.int32),
        pltpu.VMEM((OUT_CHUNK, QUANT_EMB), jnp.float32),
        pltpu.SemaphoreType.DMA,
    ],
    compiler_params=pltpu.CompilerParams(use_tc_tiling_on_sc=False),
)
def _sc_gather_out(xi_hbm, qtab_hbm, out_hbm, idx_v, rows_v, sem):
    wid = lax.axis_index("s") * 2 + lax.axis_index("c")
    nr = OUT_CHUNK // 128

    for k in range(PAIRS_PER_W):
        bc = wid * PAIRS_PER_W + k
        for c2 in range(2):
            chunk = bc * 2 + c2
            pltpu.sync_copy(xi_hbm.at[pl.ds(chunk * nr, nr)], idx_v)
            cps = [
                pltpu.async_copy(qtab_hbm.at[idx_v.at[r]],
                                 rows_v.at[pl.ds(r * 128, 128)], sem)
                for r in range(nr)
            ]
            for cp in cps:
                cp.wait()
            pltpu.sync_copy(
                rows_v,
                out_hbm.at[pl.ds(chunk * OUT_CHUNK, OUT_CHUNK)])


@functools.partial(
    pl.kernel,
    mesh=_mesh,
    out_type=jax.ShapeDtypeStruct((ROWS_COND, CLASS_EMB), jnp.float32),
    scratch_types=[
        pltpu.VMEM((COND_PER_W // 128, 128), jnp.int32),
        pltpu.VMEM((2, 128, CLASS_EMB), jnp.float32),
        pltpu.SemaphoreType.DMA,
        pltpu.SemaphoreType.DMA,
    ],
    compiler_params=pltpu.CompilerParams(use_tc_tiling_on_sc=False),
)
def _sc_gather_cond(cidx_hbm, wtab_hbm, crows_hbm, cidx_v, crows_v,
                    sem0, sem1):
    wid = lax.axis_index("s") * 2 + lax.axis_index("c")
    nr = COND_PER_W // 128
    sems = [sem0, sem1]
    pltpu.sync_copy(cidx_hbm.at[pl.ds(wid * nr, nr)], cidx_v)
    cps = [pltpu.async_copy(wtab_hbm.at[cidx_v.at[0]], crows_v.at[0], sems[0])]
    for r in range(nr):
        if r + 1 < nr:
            cps.append(pltpu.async_copy(wtab_hbm.at[cidx_v.at[r + 1]],
                                        crows_v.at[(r + 1) % 2],
                                        sems[(r + 1) % 2]))
        cps[r].wait()
        pltpu.sync_copy(crows_v.at[r % 2],
                        crows_hbm.at[pl.ds(wid * COND_PER_W + r * 128, 128)])


TB = 512


def _tc_expand_body(crows_ref, cond_ref, out_ref):
    rows = crows_ref[0]
    mask = (cond_ref[0] > 0).astype(jnp.float32)
    val = rows.T * mask
    for c in range(C):
        out_ref[c] = val


def _tc_expand(crows3, condition):
    return pl.pallas_call(
        _tc_expand_body,
        grid=(B, T // TB),
        in_specs=[
            pl.BlockSpec((1, TB, CLASS_EMB), lambda b, t: (b, t, 0)),
            pl.BlockSpec((1, 1, TB), lambda b, t: (b, 0, t)),
        ],
        out_specs=pl.BlockSpec((C, CLASS_EMB, TB), lambda b, t: (b, 0, t)),
        out_shape=jax.ShapeDtypeStruct((B * C, CLASS_EMB, T), jnp.float32),
    )(crows3, condition)


def _tc_trans_body(rows2_ref, out_ref):
    blk = rows2_ref[...]
    for c in range(C):
        sub = blk[c * (T // 2):(c + 1) * (T // 2), :]
        out_ref[c, :, :T // 2] = sub[:, :QUANT_EMB].T
        out_ref[c, :, T // 2:] = sub[:, QUANT_EMB:].T


def _tc_trans(rows2):
    return pl.pallas_call(
        _tc_trans_body,
        grid=(B,),
        in_specs=[pl.BlockSpec((C * T // 2, 128), lambda b: (b, 0))],
        out_specs=pl.BlockSpec((C, QUANT_EMB, T), lambda b: (0, 0, b)),
        out_shape=jax.ShapeDtypeStruct((C, QUANT_EMB, B * T), jnp.float32),
    )(rows2)


def kernel(x, condition, quant_emb, cond_emb_weight):
    xi = (x.reshape(B, C, 2, T // 2).swapaxes(2, 3)
          + (jnp.arange(C, dtype=x.dtype) * QUANT_LEVELS)[None, :, None, None]
          ).reshape(ROWS_OUT // 128, 128)
    cidx = condition.reshape(ROWS_COND // 128, 128)
    qtab = quant_emb.reshape(C * QUANT_LEVELS, QUANT_EMB)

    crows = _sc_gather_cond(cidx, cond_emb_weight)
    out_rows = _sc_gather_out(xi, qtab)

    cond = _tc_expand(crows.reshape(B, T, CLASS_EMB), condition)
    out_t = _tc_trans(out_rows.reshape(ROWS_OUT // 2, 2 * QUANT_EMB))
    return jnp.transpose(out_t, (2, 0, 1)), cond


# speedup vs baseline: 1.1229x; 1.1229x over previous
import functools

import jax
import jax.numpy as jnp
from jax import lax
from jax.experimental import pallas as pl
from jax.experimental.pallas import tpu as pltpu
from jax.experimental.pallas import tpu_sc as plsc

B, C, T = 16, 8, 2048
QUANT_LEVELS, QUANT_EMB = 1024, 64
NUM_CLASSES, CLASS_EMB = 1000, 128

NW = 32
ROWS_OUT = B * T * C
ROWS_COND = B * T
PAIRS_PER_W = B * C // NW
COND_PER_W = ROWS_COND // NW
OUT_CHUNK = 1024
L = 16

_mesh = plsc.VectorSubcoreMesh(core_axis_name="c", subcore_axis_name="s")


@functools.partial(
    pl.kernel,
    mesh=_mesh,
    out_type=jax.ShapeDtypeStruct((ROWS_OUT, QUANT_EMB), jnp.float32),
    scratch_types=[
        pltpu.VMEM((OUT_CHUNK // 128, 128), jnp.int32),
        pltpu.VMEM((OUT_CHUNK, QUANT_EMB), jnp.float32),
        pltpu.SemaphoreType.DMA,
    ],
    compiler_params=pltpu.CompilerParams(use_tc_tiling_on_sc=False),
)
def _sc_gather_out(xi_hbm, qtab_hbm, crows_hbm, out_hbm,
                   idx_v, rows_v, sem):
    del crows_hbm
    wid = lax.axis_index("s") * 2 + lax.axis_index("c")
    nr = OUT_CHUNK // 128

    for k in range(PAIRS_PER_W):
        bc = wid * PAIRS_PER_W + k
        for c2 in range(2):
            chunk = bc * 2 + c2
            pltpu.sync_copy(xi_hbm.at[pl.ds(chunk * nr, nr)], idx_v)
            cps = [
                pltpu.async_copy(qtab_hbm.at[idx_v.at[r]],
                                 rows_v.at[pl.ds(r * 128, 128)], sem)
                for r in range(nr)
            ]
            for cp in cps:
                cp.wait()
            pltpu.sync_copy(
                rows_v,
                out_hbm.at[pl.ds(chunk * OUT_CHUNK, OUT_CHUNK)])


@functools.partial(
    pl.kernel,
    mesh=_mesh,
    out_type=jax.ShapeDtypeStruct((ROWS_COND, CLASS_EMB), jnp.float32),
    scratch_types=[
        pltpu.VMEM((COND_PER_W // 128, 128), jnp.int32),
        pltpu.VMEM((2, 128, CLASS_EMB), jnp.float32),
        pltpu.SemaphoreType.DMA,
        pltpu.SemaphoreType.DMA,
    ],
    compiler_params=pltpu.CompilerParams(use_tc_tiling_on_sc=False),
)
def _sc_gather_cond(cidx_hbm, wtab_hbm, crows_hbm, cidx_v, crows_v,
                    sem0, sem1):
    wid = lax.axis_index("s") * 2 + lax.axis_index("c")
    nr = COND_PER_W // 128
    sems = [sem0, sem1]
    pltpu.sync_copy(cidx_hbm.at[pl.ds(wid * nr, nr)], cidx_v)
    cps = [pltpu.async_copy(wtab_hbm.at[cidx_v.at[0]], crows_v.at[0], sems[0])]
    for r in range(nr):
        if r + 1 < nr:
            cps.append(pltpu.async_copy(wtab_hbm.at[cidx_v.at[r + 1]],
                                        crows_v.at[(r + 1) % 2],
                                        sems[(r + 1) % 2]))
        cps[r].wait()
        pltpu.sync_copy(crows_v.at[r % 2],
                        crows_hbm.at[pl.ds(wid * COND_PER_W + r * 128, 128)])


TB = 512


def _tc_expand_body(crows_ref, cond_ref, out_ref):
    rows = crows_ref[0]
    mask = (cond_ref[0] > 0).astype(jnp.float32)
    val = rows.T * mask
    for c in range(C):
        out_ref[c] = val


def _tc_expand(crows3, condition):
    return pl.pallas_call(
        _tc_expand_body,
        grid=(B, T // TB),
        in_specs=[
            pl.BlockSpec((1, TB, CLASS_EMB), lambda b, t: (b, t, 0)),
            pl.BlockSpec((1, 1, TB), lambda b, t: (b, 0, t)),
        ],
        out_specs=pl.BlockSpec((C, CLASS_EMB, TB), lambda b, t: (b, 0, t)),
        out_shape=jax.ShapeDtypeStruct((B * C, CLASS_EMB, T), jnp.float32),
        compiler_params=pltpu.CompilerParams(
            dimension_semantics=("parallel", "parallel")),
    )(crows3, condition)


def _tc_trans_body(rows2_ref, out_ref):
    blk = rows2_ref[...]
    for c in range(C):
        sub = blk[c * (T // 2):(c + 1) * (T // 2), :]
        out_ref[c, :, :T // 2] = sub[:, :QUANT_EMB].T
        out_ref[c, :, T // 2:] = sub[:, QUANT_EMB:].T


def _tc_trans(rows2):
    return pl.pallas_call(
        _tc_trans_body,
        grid=(B,),
        in_specs=[pl.BlockSpec((C * T // 2, 128), lambda b: (b, 0))],
        out_specs=pl.BlockSpec((C, QUANT_EMB, T), lambda b: (0, 0, b)),
        out_shape=jax.ShapeDtypeStruct((C, QUANT_EMB, B * T), jnp.float32),
        compiler_params=pltpu.CompilerParams(
            dimension_semantics=("parallel",)),
    )(rows2)


def kernel(x, condition, quant_emb, cond_emb_weight):
    xi = (x.reshape(B, C, 2, T // 2).swapaxes(2, 3)
          + (jnp.arange(C, dtype=x.dtype) * QUANT_LEVELS)[None, :, None, None]
          ).reshape(ROWS_OUT // 128, 128)
    cidx = condition.reshape(ROWS_COND // 128, 128)
    qtab = quant_emb.reshape(C * QUANT_LEVELS, QUANT_EMB)

    crows = _sc_gather_cond(cidx, cond_emb_weight)
    out_rows = _sc_gather_out(xi, qtab, crows)

    cond = _tc_expand(crows.reshape(B, T, CLASS_EMB), condition)
    out_t = _tc_trans(out_rows.reshape(ROWS_OUT // 2, 2 * QUANT_EMB))
    return jnp.transpose(out_t, (2, 0, 1)), cond


# speedup vs baseline: 1.4488x; 1.2902x over previous
import functools

import jax
import jax.numpy as jnp
from jax import lax
from jax.experimental import pallas as pl
from jax.experimental.pallas import tpu as pltpu
from jax.experimental.pallas import tpu_sc as plsc

B, C, T = 16, 8, 2048
QUANT_LEVELS, QUANT_EMB = 1024, 64
NUM_CLASSES, CLASS_EMB = 1000, 128

NW = 32
ROWS_OUT = B * T * C
ROWS_COND = B * T
PAIRS_PER_W = B * C // NW
COND_PER_W = ROWS_COND // NW
OUT_CHUNK = 1024
L = 16

_mesh = plsc.VectorSubcoreMesh(core_axis_name="c", subcore_axis_name="s")


@functools.partial(
    pl.kernel,
    mesh=_mesh,
    out_type=jax.ShapeDtypeStruct((B * C, T // 2, 2, QUANT_EMB), jnp.float32),
    scratch_types=[
        pltpu.VMEM((OUT_CHUNK // 128, 128), jnp.int32),
        pltpu.VMEM((OUT_CHUNK, QUANT_EMB), jnp.float32),
        pltpu.SemaphoreType.DMA,
    ],
    compiler_params=pltpu.CompilerParams(use_tc_tiling_on_sc=False),
)
def _sc_gather_out(xi_hbm, qtab_hbm, crows_hbm, out_hbm,
                   idx_v, rows_v, sem):
    del crows_hbm
    wid = lax.axis_index("s") * 2 + lax.axis_index("c")
    nr = OUT_CHUNK // 128

    for k in range(PAIRS_PER_W):
        bc = wid * PAIRS_PER_W + k
        for c2 in range(2):
            chunk = bc * 2 + c2
            pltpu.sync_copy(xi_hbm.at[pl.ds(chunk * nr, nr)], idx_v)
            cps = [
                pltpu.async_copy(qtab_hbm.at[idx_v.at[r]],
                                 rows_v.at[pl.ds(r * 128, 128)], sem)
                for r in range(nr)
            ]
            for cp in cps:
                cp.wait()
            pltpu.sync_copy(rows_v, out_hbm.at[bc, :, c2])


@functools.partial(
    pl.kernel,
    mesh=_mesh,
    out_type=jax.ShapeDtypeStruct((ROWS_COND, CLASS_EMB), jnp.float32),
    scratch_types=[
        pltpu.VMEM((COND_PER_W // 128, 128), jnp.int32),
        pltpu.VMEM((2, 128, CLASS_EMB), jnp.float32),
        pltpu.SemaphoreType.DMA,
        pltpu.SemaphoreType.DMA,
    ],
    compiler_params=pltpu.CompilerParams(use_tc_tiling_on_sc=False),
)
def _sc_gather_cond(cidx_hbm, wtab_hbm, crows_hbm, cidx_v, crows_v,
                    sem0, sem1):
    wid = lax.axis_index("s") * 2 + lax.axis_index("c")
    nr = COND_PER_W // 128
    sems = [sem0, sem1]
    pltpu.sync_copy(cidx_hbm.at[pl.ds(wid * nr, nr)], cidx_v)
    cps = [pltpu.async_copy(wtab_hbm.at[cidx_v.at[0]], crows_v.at[0], sems[0])]
    for r in range(nr):
        if r + 1 < nr:
            cps.append(pltpu.async_copy(wtab_hbm.at[cidx_v.at[r + 1]],
                                        crows_v.at[(r + 1) % 2],
                                        sems[(r + 1) % 2]))
        cps[r].wait()
        pltpu.sync_copy(crows_v.at[r % 2],
                        crows_hbm.at[pl.ds(wid * COND_PER_W + r * 128, 128)])


TB = 512


def _tc_expand_body(crows_ref, cond_ref, out_ref):
    rows = crows_ref[0]
    mask = (cond_ref[0] > 0).astype(jnp.float32)
    val = rows.T * mask
    for c in range(C):
        out_ref[c] = val


def _tc_expand(crows3, condition):
    return pl.pallas_call(
        _tc_expand_body,
        grid=(B, T // TB),
        in_specs=[
            pl.BlockSpec((1, TB, CLASS_EMB), lambda b, t: (b, t, 0)),
            pl.BlockSpec((1, 1, TB), lambda b, t: (b, 0, t)),
        ],
        out_specs=pl.BlockSpec((C, CLASS_EMB, TB), lambda b, t: (b, 0, t)),
        out_shape=jax.ShapeDtypeStruct((B * C, CLASS_EMB, T), jnp.float32),
        compiler_params=pltpu.CompilerParams(
            dimension_semantics=("parallel", "parallel")),
    )(crows3, condition)


def _tc_trans_body(rows2_ref, out_ref):
    blk = rows2_ref[...]
    for c in range(C):
        sub = blk[c * (T // 2):(c + 1) * (T // 2), :]
        out_ref[c, :, :T // 2] = sub[:, :QUANT_EMB].T
        out_ref[c, :, T // 2:] = sub[:, QUANT_EMB:].T


def _tc_trans(rows2):
    return pl.pallas_call(
        _tc_trans_body,
        grid=(B,),
        in_specs=[pl.BlockSpec((C * T // 2, 128), lambda b: (b, 0))],
        out_specs=pl.BlockSpec((C, QUANT_EMB, T), lambda b: (0, 0, b)),
        out_shape=jax.ShapeDtypeStruct((C, QUANT_EMB, B * T), jnp.float32),
        compiler_params=pltpu.CompilerParams(
            dimension_semantics=("parallel",)),
    )(rows2)


def kernel(x, condition, quant_emb, cond_emb_weight):
    nrow = ROWS_OUT // 128
    off = ((jnp.arange(nrow, dtype=x.dtype) // 16) % C) * QUANT_LEVELS
    xi = x.reshape(nrow, 128) + off[:, None]
    cidx = condition.reshape(ROWS_COND // 128, 128)
    qtab = quant_emb.reshape(C * QUANT_LEVELS, QUANT_EMB)

    crows = _sc_gather_cond(cidx, cond_emb_weight)
    out_rows = _sc_gather_out(xi, qtab, crows)

    cond = _tc_expand(crows.reshape(B, T, CLASS_EMB), condition)
    out_t = _tc_trans(out_rows.reshape(ROWS_OUT // 2, 2 * QUANT_EMB))
    return jnp.transpose(out_t, (2, 0, 1)), cond
